# baseline (speedup 1.0000x reference)
import functools

import jax
import jax.numpy as jnp
from jax import lax
from jax.experimental import pallas as pl
from jax.experimental.pallas import tpu as pltpu
from jax.experimental.pallas import tpu_sc as plsc

_NC = 2
_NS = 16
_NW = _NC * _NS
_CHUNK = 128
_LANES = 128
_NB = 4


def _mm1(x, w1):
    n, d_in = x.shape
    d_hid = w1.shape[1]
    blk = 2000
    grid = n // blk

    def body(x_ref, w_ref, o_ref):
        res = jnp.dot(x_ref[...], w_ref[...],
                      preferred_element_type=jnp.float32)
        o_ref[...] = jnp.pad(res, ((0, 0), (0, _LANES - d_hid)))

    return pl.pallas_call(
        body,
        grid=(grid,),
        in_specs=[
            pl.BlockSpec((blk, d_in), lambda i: (i, 0)),
            pl.BlockSpec((d_in, d_hid), lambda i: (0, 0)),
        ],
        out_specs=pl.BlockSpec((blk, _LANES), lambda i: (i, 0)),
        out_shape=jax.ShapeDtypeStruct((n, _LANES), jnp.float32),
    )(x, w1)


def _sc_gather_scatter_add(edges, h, zrows, d_hid, stripe, kbase, kextra, kmax):
    n = h.shape[0]
    h_stripe = n // _NS
    mesh = plsc.VectorSubcoreMesh(core_axis_name="c", subcore_axis_name="s")

    @functools.partial(
        pl.kernel,
        mesh=mesh,
        compiler_params=pltpu.CompilerParams(use_tc_tiling_on_sc=False),
        out_type=jax.ShapeDtypeStruct((_NC, n, _LANES), jnp.float32),
        scratch_types=[
            pltpu.VMEM((kmax, _CHUNK), jnp.int32),
            pltpu.VMEM((kmax, _CHUNK), jnp.int32),
            pltpu.VMEM((_NB, _CHUNK, d_hid), jnp.float32),
            pltpu.VMEM_SHARED((n, d_hid), jnp.float32),
            pltpu.VMEM_SHARED((_NS * (n // _NS), d_hid), jnp.float32),
        ] + [pltpu.SemaphoreType.DMA] * (2 * _NB),
    )
    def k(e_hbm, h_hbm, z_hbm, out_hbm,
          sidx, didx, rowsb, hsh, acc, *sems):
        gs = sems[:_NB]
        ss = sems[_NB:]
        c = lax.axis_index("c")
        s = lax.axis_index("s")
        g = c * _NS + s
        kw = kbase + jnp.where(g < kextra, 1, 0)
        cb = g * kbase + jnp.minimum(g, kextra)
        pltpu.sync_copy(z_hbm, acc.at[pl.ds(s * stripe, stripe)])
        pltpu.sync_copy(h_hbm.at[pl.ds(s * h_stripe, h_stripe), pl.ds(0, d_hid)],
                        hsh.at[pl.ds(s * h_stripe, h_stripe)])
        pltpu.sync_copy(e_hbm.at[0, pl.ds(cb, kbase)], sidx.at[pl.ds(0, kbase)])
        pltpu.sync_copy(e_hbm.at[1, pl.ds(cb, kbase)], didx.at[pl.ds(0, kbase)])

        @pl.when(g < kextra)
        def _():
            pltpu.sync_copy(e_hbm.at[0, pl.ds(cb + kbase, 1)],
                            sidx.at[pl.ds(kbase, 1)])
            pltpu.sync_copy(e_hbm.at[1, pl.ds(cb + kbase, 1)],
                            didx.at[pl.ds(kbase, 1)])

        plsc.subcore_barrier()

        for b in range(_NB):
            @pl.when(b < kw)
            def _(b=b):
                pltpu.async_copy(hsh.at[sidx.at[b]], rowsb.at[b], gs[b])

        @pl.loop(0, kw, step=_NB)
        def _(j):
            for b in range(_NB):
                @pl.when(j + b < kw)
                def _(b=b):
                    pltpu.make_async_copy(hsh.at[sidx.at[j + b]],
                                          rowsb.at[b], gs[b]).wait()
                    pltpu.async_copy(rowsb.at[b], acc.at[didx.at[j + b]],
                                     ss[b], add=True)
            for b in range(_NB):
                @pl.when(j + b + _NB < kw)
                def _(b=b):
                    pltpu.make_async_copy(rowsb.at[b], acc.at[didx.at[j + b]],
                                          ss[b]).wait()
                    pltpu.async_copy(hsh.at[sidx.at[j + b + _NB]],
                                     rowsb.at[b], gs[b])

        for b in range(_NB):
            pltpu.make_async_copy(rowsb.at[b], acc.at[didx.at[0]], ss[b]).wait()

        plsc.subcore_barrier()
        pltpu.sync_copy(acc.at[pl.ds(s * stripe, stripe)],
                        out_hbm.at[c, pl.ds(s * stripe, stripe), pl.ds(0, d_hid)])

    return k(edges, h, zrows)


def _tc2(part, b1r, w2, n, d_hid):
    d_out = w2.shape[1]
    blk = 2000
    grid = n // blk

    def body(p0_ref, p1_ref, b_ref, w_ref, h1_ref, h2_ref):
        acc = (p0_ref[0, :, :d_hid] + p1_ref[0, :, :d_hid]) + b_ref[...]
        h1_ref[...] = acc
        h2_ref[...] = jnp.dot(acc, w_ref[...],
                              preferred_element_type=jnp.float32)

    return pl.pallas_call(
        body,
        grid=(grid,),
        in_specs=[
            pl.BlockSpec((1, blk, _LANES), lambda i: (0, i, 0)),
            pl.BlockSpec((1, blk, _LANES), lambda i: (1, i, 0)),
            pl.BlockSpec((1, d_hid), lambda i: (0, 0)),
            pl.BlockSpec((d_hid, d_out), lambda i: (0, 0)),
        ],
        out_specs=[
            pl.BlockSpec((blk, d_hid), lambda i: (i, 0)),
            pl.BlockSpec((blk, d_out), lambda i: (i, 0)),
        ],
        out_shape=[
            jax.ShapeDtypeStruct((n, d_hid), jnp.float32),
            jax.ShapeDtypeStruct((n, d_out), jnp.float32),
        ],
    )(part, part, b1r, w2)


def kernel(x, edge_index, W1, b1, W2):
    n, d_hid = x.shape[0], W1.shape[1]
    e = edge_index.shape[1]

    total_chunks = e // _CHUNK
    kbase = total_chunks // _NW
    kextra = total_chunks % _NW
    kmax = kbase + (1 if kextra else 0)
    stripe = n // _NS

    edges = edge_index.reshape(2, total_chunks, _CHUNK)
    zrows = jnp.zeros((stripe, d_hid), jnp.float32)

    h = _mm1(x, W1)
    part = _sc_gather_scatter_add(edges, h, zrows, d_hid,
                                  stripe, kbase, kextra, kmax)
    h1, h2 = _tc2(part, b1.reshape(1, d_hid), W2, n, d_hid)
    return (h1, h2)


# speedup vs baseline: 1.0724x; 1.0724x over previous
import functools

import jax
import jax.numpy as jnp
from jax import lax
from jax.experimental import pallas as pl
from jax.experimental.pallas import tpu as pltpu
from jax.experimental.pallas import tpu_sc as plsc

_NC = 2
_NS = 16
_NW = _NC * _NS
_CHUNK = 128
_LANES = 128
_NB = 2


def _mm1(x, w1, edge_index):
    n, d_in = x.shape
    d_hid = w1.shape[1]
    e = edge_index.shape[1]
    blk = 2000
    grid = n // blk
    eblk = e // grid

    def body(x_ref, w_ref, e_ref, o_ref, os_ref, od_ref):
        i = pl.program_id(0)
        res = jnp.dot(x_ref[...], w_ref[...],
                      preferred_element_type=jnp.float32)
        o_ref[...] = jnp.pad(res, ((0, 0), (0, _LANES - d_hid)))
        os_ref[pl.ds(i * eblk, eblk)] = e_ref[0]
        od_ref[pl.ds(i * eblk, eblk)] = e_ref[1]

    return pl.pallas_call(
        body,
        grid=(grid,),
        in_specs=[
            pl.BlockSpec((blk, d_in), lambda i: (i, 0)),
            pl.BlockSpec((d_in, d_hid), lambda i: (0, 0)),
            pl.BlockSpec((2, eblk), lambda i: (0, i)),
        ],
        out_specs=[
            pl.BlockSpec((blk, _LANES), lambda i: (i, 0)),
            pl.BlockSpec((e,), lambda i: (0,)),
            pl.BlockSpec((e,), lambda i: (0,)),
        ],
        out_shape=[
            jax.ShapeDtypeStruct((n, _LANES), jnp.float32),
            jax.ShapeDtypeStruct((e,), jnp.int32),
            jax.ShapeDtypeStruct((e,), jnp.int32),
        ],
    )(x, w1, edge_index)


def _sc_gather_scatter_add(src1d, dst1d, h, zrows, d_hid, stripe,
                           kbase, kextra, kmax):
    n = h.shape[0]
    h_stripe = n // _NS
    mesh = plsc.VectorSubcoreMesh(core_axis_name="c", subcore_axis_name="s")

    @functools.partial(
        pl.kernel,
        mesh=mesh,
        compiler_params=pltpu.CompilerParams(use_tc_tiling_on_sc=False),
        out_type=jax.ShapeDtypeStruct((_NC, n, _LANES), jnp.float32),
        scratch_types=[
            pltpu.VMEM((kmax * _CHUNK,), jnp.int32),
            pltpu.VMEM((kmax * _CHUNK,), jnp.int32),
            pltpu.VMEM((_NB, _CHUNK, d_hid), jnp.float32),
            pltpu.VMEM_SHARED((n, d_hid), jnp.float32),
            pltpu.VMEM_SHARED((_NS * (n // _NS), d_hid), jnp.float32),
        ] + [pltpu.SemaphoreType.DMA] * (2 * _NB),
    )
    def k(s_hbm, d_hbm, h_hbm, z_hbm, out_hbm,
          sidx, didx, rowsb, hsh, acc, *sems):
        gs = sems[:_NB]
        ss = sems[_NB:]
        c = lax.axis_index("c")
        s = lax.axis_index("s")
        g = c * _NS + s
        kw = kbase + jnp.where(g < kextra, 1, 0)
        cb = g * kbase + jnp.minimum(g, kextra)
        pltpu.sync_copy(z_hbm, acc.at[pl.ds(s * stripe, stripe)])
        pltpu.sync_copy(h_hbm.at[pl.ds(s * h_stripe, h_stripe), pl.ds(0, d_hid)],
                        hsh.at[pl.ds(s * h_stripe, h_stripe)])
        pltpu.sync_copy(s_hbm.at[pl.ds(cb * _CHUNK, kbase * _CHUNK)],
                        sidx.at[pl.ds(0, kbase * _CHUNK)])
        pltpu.sync_copy(d_hbm.at[pl.ds(cb * _CHUNK, kbase * _CHUNK)],
                        didx.at[pl.ds(0, kbase * _CHUNK)])

        @pl.when(g < kextra)
        def _():
            pltpu.sync_copy(s_hbm.at[pl.ds((cb + kbase) * _CHUNK, _CHUNK)],
                            sidx.at[pl.ds(kbase * _CHUNK, _CHUNK)])
            pltpu.sync_copy(d_hbm.at[pl.ds((cb + kbase) * _CHUNK, _CHUNK)],
                            didx.at[pl.ds(kbase * _CHUNK, _CHUNK)])

        plsc.subcore_barrier()

        for b in range(_NB):
            @pl.when(b < kw)
            def _(b=b):
                pltpu.async_copy(hsh.at[sidx.at[pl.ds(b * _CHUNK, _CHUNK)]],
                                 rowsb.at[b], gs[b])

        @pl.loop(0, kw, step=_NB)
        def _(j):
            for b in range(_NB):
                @pl.when(j + b < kw)
                def _(b=b):
                    pltpu.make_async_copy(
                        hsh.at[sidx.at[pl.ds((j + b) * _CHUNK, _CHUNK)]],
                        rowsb.at[b], gs[b]).wait()
                    pltpu.async_copy(rowsb.at[b], acc.at[didx.at[pl.ds((j + b) * _CHUNK, _CHUNK)]],
                                     ss[b], add=True)
            for b in range(_NB):
                @pl.when(j + b + _NB < kw)
                def _(b=b):
                    pltpu.make_async_copy(rowsb.at[b], acc.at[didx.at[pl.ds((j + b) * _CHUNK, _CHUNK)]],
                                          ss[b]).wait()
                    pltpu.async_copy(
                        hsh.at[sidx.at[pl.ds((j + b + _NB) * _CHUNK, _CHUNK)]],
                        rowsb.at[b], gs[b])

        for b in range(_NB):
            pltpu.make_async_copy(rowsb.at[b], acc.at[didx.at[pl.ds(0, _CHUNK)]], ss[b]).wait()

        plsc.subcore_barrier()
        pltpu.sync_copy(acc.at[pl.ds(s * stripe, stripe)],
                        out_hbm.at[c, pl.ds(s * stripe, stripe), pl.ds(0, d_hid)])

    return k(src1d, dst1d, h, zrows)


def _tc2(part, b1r, w2, n, d_hid):
    d_out = w2.shape[1]
    blk = 2000
    grid = n // blk

    def body(p0_ref, p1_ref, b_ref, w_ref, h1_ref, h2_ref):
        acc = (p0_ref[0, :, :d_hid] + p1_ref[0, :, :d_hid]) + b_ref[...]
        h1_ref[...] = acc
        h2_ref[...] = jnp.dot(acc, w_ref[...],
                              preferred_element_type=jnp.float32)

    return pl.pallas_call(
        body,
        grid=(grid,),
        in_specs=[
            pl.BlockSpec((1, blk, _LANES), lambda i: (0, i, 0)),
            pl.BlockSpec((1, blk, _LANES), lambda i: (1, i, 0)),
            pl.BlockSpec((1, d_hid), lambda i: (0, 0)),
            pl.BlockSpec((d_hid, d_out), lambda i: (0, 0)),
        ],
        out_specs=[
            pl.BlockSpec((blk, d_hid), lambda i: (i, 0)),
            pl.BlockSpec((blk, d_out), lambda i: (i, 0)),
        ],
        out_shape=[
            jax.ShapeDtypeStruct((n, d_hid), jnp.float32),
            jax.ShapeDtypeStruct((n, d_out), jnp.float32),
        ],
    )(part, part, b1r, w2)


def kernel(x, edge_index, W1, b1, W2):
    n, d_hid = x.shape[0], W1.shape[1]
    e = edge_index.shape[1]

    total_chunks = e // _CHUNK
    kbase = total_chunks // _NW
    kextra = total_chunks % _NW
    kmax = kbase + (1 if kextra else 0)
    stripe = n // _NS

    zrows = jnp.zeros((stripe, d_hid), jnp.float32)

    h, src1d, dst1d = _mm1(x, W1, edge_index)
    part = _sc_gather_scatter_add(src1d, dst1d, h, zrows, d_hid,
                                  stripe, kbase, kextra, kmax)
    h1, h2 = _tc2(part, b1.reshape(1, d_hid), W2, n, d_hid)
    return (h1, h2)


# speedup vs baseline: 1.0729x; 1.0004x over previous
import functools

import jax
import jax.numpy as jnp
from jax import lax
from jax.experimental import pallas as pl
from jax.experimental.pallas import tpu as pltpu
from jax.experimental.pallas import tpu_sc as plsc

_NC = 2
_NS = 16
_NW = _NC * _NS
_CHUNK = 128
_LANES = 128
_NB = 2


def _mm1(x, w1, edge_index):
    n, d_in = x.shape
    d_hid = w1.shape[1]
    e = edge_index.shape[1]
    blk = 2000
    grid = n // blk
    eblk = e // grid

    def body(x_ref, w_ref, e_ref, o_ref, os_ref, od_ref):
        i = pl.program_id(0)
        res = jnp.dot(x_ref[...], w_ref[...],
                      preferred_element_type=jnp.float32)
        o_ref[...] = jnp.pad(res, ((0, 0), (0, _LANES - d_hid)))
        os_ref[pl.ds(i * eblk, eblk)] = e_ref[0]
        od_ref[pl.ds(i * eblk, eblk)] = e_ref[1]

    return pl.pallas_call(
        body,
        grid=(grid,),
        in_specs=[
            pl.BlockSpec((blk, d_in), lambda i: (i, 0)),
            pl.BlockSpec((d_in, d_hid), lambda i: (0, 0)),
            pl.BlockSpec((2, eblk), lambda i: (0, i)),
        ],
        out_specs=[
            pl.BlockSpec((blk, _LANES), lambda i: (i, 0)),
            pl.BlockSpec((e,), lambda i: (0,)),
            pl.BlockSpec((e,), lambda i: (0,)),
        ],
        out_shape=[
            jax.ShapeDtypeStruct((n, _LANES), jnp.float32),
            jax.ShapeDtypeStruct((e,), jnp.int32),
            jax.ShapeDtypeStruct((e,), jnp.int32),
        ],
    )(x, w1, edge_index)


def _sc_gather_scatter_add(src1d, dst1d, h, d_hid, stripe,
                           kbase, kextra, kmax):
    n = h.shape[0]
    h_stripe = n // _NS
    mesh = plsc.VectorSubcoreMesh(core_axis_name="c", subcore_axis_name="s")

    @functools.partial(
        pl.kernel,
        mesh=mesh,
        compiler_params=pltpu.CompilerParams(use_tc_tiling_on_sc=False),
        out_type=jax.ShapeDtypeStruct((_NC, n, _LANES), jnp.float32),
        scratch_types=[
            pltpu.VMEM((kmax * _CHUNK,), jnp.int32),
            pltpu.VMEM((kmax * _CHUNK,), jnp.int32),
            pltpu.VMEM((_NB, _CHUNK, d_hid), jnp.float32),
            pltpu.VMEM_SHARED((n, d_hid), jnp.float32),
            pltpu.VMEM_SHARED((_NS * (n // _NS), d_hid), jnp.float32),
        ] + [pltpu.SemaphoreType.DMA] * (2 * _NB),
    )
    def k(s_hbm, d_hbm, h_hbm, out_hbm,
          sidx, didx, rowsb, hsh, acc, *sems):
        gs = sems[:_NB]
        ss = sems[_NB:]
        c = lax.axis_index("c")
        s = lax.axis_index("s")
        g = c * _NS + s
        kw = kbase + jnp.where(g < kextra, 1, 0)
        cb = g * kbase + jnp.minimum(g, kextra)
        zn = rowsb.shape[1]

        @pl.loop(0, zn)
        def _(i):
            rowsb.at[0, i, pl.ds(0, 16)][...] = jnp.zeros((16,), jnp.float32)
            rowsb.at[0, i, pl.ds(16, 16)][...] = jnp.zeros((16,), jnp.float32)

        r = 0
        while r < stripe:
            nr = min(stripe - r, zn)
            pltpu.sync_copy(rowsb.at[0, pl.ds(0, nr)],
                            acc.at[pl.ds(s * stripe + r, nr)])
            r += nr
        pltpu.sync_copy(h_hbm.at[pl.ds(s * h_stripe, h_stripe), pl.ds(0, d_hid)],
                        hsh.at[pl.ds(s * h_stripe, h_stripe)])
        pltpu.sync_copy(s_hbm.at[pl.ds(cb * _CHUNK, kbase * _CHUNK)],
                        sidx.at[pl.ds(0, kbase * _CHUNK)])
        pltpu.sync_copy(d_hbm.at[pl.ds(cb * _CHUNK, kbase * _CHUNK)],
                        didx.at[pl.ds(0, kbase * _CHUNK)])

        @pl.when(g < kextra)
        def _():
            pltpu.sync_copy(s_hbm.at[pl.ds((cb + kbase) * _CHUNK, _CHUNK)],
                            sidx.at[pl.ds(kbase * _CHUNK, _CHUNK)])
            pltpu.sync_copy(d_hbm.at[pl.ds((cb + kbase) * _CHUNK, _CHUNK)],
                            didx.at[pl.ds(kbase * _CHUNK, _CHUNK)])

        plsc.subcore_barrier()

        for b in range(_NB):
            @pl.when(b < kw)
            def _(b=b):
                pltpu.async_copy(hsh.at[sidx.at[pl.ds(b * _CHUNK, _CHUNK)]],
                                 rowsb.at[b], gs[b])

        @pl.loop(0, kw, step=_NB)
        def _(j):
            for b in range(_NB):
                @pl.when(j + b < kw)
                def _(b=b):
                    pltpu.make_async_copy(
                        hsh.at[sidx.at[pl.ds((j + b) * _CHUNK, _CHUNK)]],
                        rowsb.at[b], gs[b]).wait()
                    pltpu.async_copy(rowsb.at[b], acc.at[didx.at[pl.ds((j + b) * _CHUNK, _CHUNK)]],
                                     ss[b], add=True)
            for b in range(_NB):
                @pl.when(j + b + _NB < kw)
                def _(b=b):
                    pltpu.make_async_copy(rowsb.at[b], acc.at[didx.at[pl.ds((j + b) * _CHUNK, _CHUNK)]],
                                          ss[b]).wait()
                    pltpu.async_copy(
                        hsh.at[sidx.at[pl.ds((j + b + _NB) * _CHUNK, _CHUNK)]],
                        rowsb.at[b], gs[b])

        for b in range(_NB):
            pltpu.make_async_copy(rowsb.at[b], acc.at[didx.at[pl.ds(0, _CHUNK)]], ss[b]).wait()

        plsc.subcore_barrier()
        pltpu.sync_copy(acc.at[pl.ds(s * stripe, stripe)],
                        out_hbm.at[c, pl.ds(s * stripe, stripe), pl.ds(0, d_hid)])

    return k(src1d, dst1d, h)


def _tc2(part, b1r, w2, n, d_hid):
    d_out = w2.shape[1]
    blk = 2000
    grid = n // blk

    def body(p0_ref, p1_ref, b_ref, w_ref, h1_ref, h2_ref):
        acc = (p0_ref[0, :, :d_hid] + p1_ref[0, :, :d_hid]) + b_ref[...]
        h1_ref[...] = acc
        h2_ref[...] = jnp.dot(acc, w_ref[...],
                              preferred_element_type=jnp.float32)

    return pl.pallas_call(
        body,
        grid=(grid,),
        in_specs=[
            pl.BlockSpec((1, blk, _LANES), lambda i: (0, i, 0)),
            pl.BlockSpec((1, blk, _LANES), lambda i: (1, i, 0)),
            pl.BlockSpec((1, d_hid), lambda i: (0, 0)),
            pl.BlockSpec((d_hid, d_out), lambda i: (0, 0)),
        ],
        out_specs=[
            pl.BlockSpec((blk, d_hid), lambda i: (i, 0)),
            pl.BlockSpec((blk, d_out), lambda i: (i, 0)),
        ],
        out_shape=[
            jax.ShapeDtypeStruct((n, d_hid), jnp.float32),
            jax.ShapeDtypeStruct((n, d_out), jnp.float32),
        ],
    )(part, part, b1r, w2)


def kernel(x, edge_index, W1, b1, W2):
    n, d_hid = x.shape[0], W1.shape[1]
    e = edge_index.shape[1]

    total_chunks = e // _CHUNK
    kbase = total_chunks // _NW
    kextra = total_chunks % _NW
    kmax = kbase + (1 if kextra else 0)
    stripe = n // _NS

    h, src1d, dst1d = _mm1(x, W1, edge_index)
    part = _sc_gather_scatter_add(src1d, dst1d, h, d_hid,
                                  stripe, kbase, kextra, kmax)
    h1, h2 = _tc2(part, b1.reshape(1, d_hid), W2, n, d_hid)
    return (h1, h2)
